# Initial kernel scaffold; baseline (speedup 1.0000x reference)
import jax
import jax.numpy as jnp
from jax.experimental import pallas as pl


def kernel(x, tok_table, pos_W, pos_b, grid):
    raise NotImplementedError("write your pallas kernel here")


# speedup vs baseline: 4.5149x; 4.5149x over previous
import jax
import jax.numpy as jnp
from jax import lax
from jax.experimental import pallas as pl
from jax.experimental.pallas import tpu as pltpu
from jax.experimental.pallas import tpu_sc as plsc

_B, _N, _D = 1024, 1024, 32
_NC, _NS = 2, 16
_NW = _NC * _NS
_BLOCKS_PER_W = _B // _NW
_IDX_MINOR = 128
_JP = _N // _IDX_MINOR


def _pos_body(g_ref, w_ref, b_ref, o_ref):
    o_ref[...] = (
        jnp.dot(g_ref[...], w_ref[...], preferred_element_type=jnp.float32)
        + b_ref[...]
    )


def _sc_body(x_hbm, tab_hbm, pos_hbm, out_hbm, idx_v, rows_v, pos_v, gsem):
    c = lax.axis_index("c")
    s = lax.axis_index("s")
    wid = s * _NC + c
    pltpu.sync_copy(pos_hbm, pos_v)

    def chunk(t, carry):
        blk = wid * _BLOCKS_PER_W + t
        pltpu.sync_copy(x_hbm.at[blk], idx_v)
        cps = [
            pltpu.async_copy(
                tab_hbm.at[idx_v.at[j]],
                rows_v.at[pl.ds(j * _IDX_MINOR, _IDX_MINOR)],
                gsem,
            )
            for j in range(_JP)
        ]
        for cp in cps:
            cp.wait()

        def add_row(r, carry2):
            for h in range(2):
                sl = pl.ds(h * 16, 16)
                rows_v[r, sl] = rows_v[r, sl] + pos_v[r, sl]
            return carry2

        lax.fori_loop(0, _N, add_row, 0)
        pltpu.sync_copy(rows_v, out_hbm.at[blk])
        return carry

    lax.fori_loop(0, _BLOCKS_PER_W, chunk, 0)


def kernel(x, tok_table, pos_W, pos_b, grid):
    g2 = grid.reshape(_N, 4)
    pos = pl.pallas_call(
        _pos_body,
        out_shape=jax.ShapeDtypeStruct((_N, _D), jnp.float32),
    )(g2, pos_W.T, pos_b.reshape(1, _D))

    x3 = x.reshape(_B, _JP, _IDX_MINOR)
    sc = pl.kernel(
        _sc_body,
        out_type=jax.ShapeDtypeStruct((_B, _N, _D), jnp.float32),
        mesh=plsc.VectorSubcoreMesh(core_axis_name="c", subcore_axis_name="s"),
        compiler_params=pltpu.CompilerParams(use_tc_tiling_on_sc=False),
        scratch_types=[
            pltpu.VMEM((_JP, _IDX_MINOR), jnp.int32),
            pltpu.VMEM((_N, _D), jnp.float32),
            pltpu.VMEM((_N, _D), jnp.float32),
            pltpu.SemaphoreType.DMA,
        ],
    )
    return sc(x3, tok_table, pos)


# speedup vs baseline: 5.4126x; 1.1988x over previous
import jax
import jax.numpy as jnp
from jax import lax
from jax.experimental import pallas as pl
from jax.experimental.pallas import tpu as pltpu
from jax.experimental.pallas import tpu_sc as plsc

_B, _N, _D = 1024, 1024, 32
_NC, _NS = 2, 16
_NW = _NC * _NS
_BLOCKS_PER_W = _B // _NW
_IDX_MINOR = 128
_JP = _N // _IDX_MINOR


def _pos_body(g_ref, w_ref, b_ref, o_ref):
    o_ref[...] = (
        jnp.dot(g_ref[...], w_ref[...], preferred_element_type=jnp.float32)
        + b_ref[...]
    )


def _sc_body(x_hbm, tab_hbm, pos_hbm, out_hbm,
             idx0, idx1, rows0, rows1, pos_v, g0, g1, o0, o1):
    c = lax.axis_index("c")
    s = lax.axis_index("s")
    wid = s * _NC + c
    base = wid * _BLOCKS_PER_W
    idx = (idx0, idx1)
    rows = (rows0, rows1)
    gsem = (g0, g1)
    osem = (o0, o1)
    pltpu.sync_copy(pos_hbm, pos_v)

    def fire(t, buf):
        pltpu.sync_copy(x_hbm.at[base + t], idx[buf])
        for j in range(_JP):
            pltpu.async_copy(
                tab_hbm.at[idx[buf].at[j]],
                rows[buf].at[pl.ds(j * _IDX_MINOR, _IDX_MINOR)],
                gsem[buf],
            )

    fire(0, 0)
    for t in range(_BLOCKS_PER_W):
        cur = t % 2
        nxt = 1 - cur
        if t + 1 < _BLOCKS_PER_W:
            if t >= 1:
                pltpu.make_async_copy(rows[nxt], out_hbm.at[base], osem[nxt]).wait()
            fire(t + 1, nxt)
        pltpu.make_async_copy(out_hbm.at[base], rows[cur], gsem[cur]).wait()

        @plsc.parallel_loop(0, _N, step=1, unroll=8)
        def add_row(r):
            for h in range(2):
                sl = pl.ds(h * 16, 16)
                rows[cur][r, sl] = rows[cur][r, sl] + pos_v[r, sl]

        pltpu.async_copy(rows[cur], out_hbm.at[base + t], osem[cur])

    pltpu.make_async_copy(rows[0], out_hbm.at[base], osem[0]).wait()
    pltpu.make_async_copy(rows[1], out_hbm.at[base], osem[1]).wait()


def kernel(x, tok_table, pos_W, pos_b, grid):
    g2 = grid.reshape(_N, 4)
    pos = pl.pallas_call(
        _pos_body,
        out_shape=jax.ShapeDtypeStruct((_N, _D), jnp.float32),
    )(g2, pos_W.T, pos_b.reshape(1, _D))

    x3 = x.reshape(_B, _JP, _IDX_MINOR)
    sc = pl.kernel(
        _sc_body,
        out_type=jax.ShapeDtypeStruct((_B, _N, _D), jnp.float32),
        mesh=plsc.VectorSubcoreMesh(core_axis_name="c", subcore_axis_name="s"),
        compiler_params=pltpu.CompilerParams(use_tc_tiling_on_sc=False),
        scratch_types=[
            pltpu.VMEM((_JP, _IDX_MINOR), jnp.int32),
            pltpu.VMEM((_JP, _IDX_MINOR), jnp.int32),
            pltpu.VMEM((_N, _D), jnp.float32),
            pltpu.VMEM((_N, _D), jnp.float32),
            pltpu.VMEM((_N, _D), jnp.float32),
            pltpu.SemaphoreType.DMA,
            pltpu.SemaphoreType.DMA,
            pltpu.SemaphoreType.DMA,
            pltpu.SemaphoreType.DMA,
        ],
    )
    return sc(x3, tok_table, pos)
